# baseline (speedup 1.0000x reference)
import functools

import jax
import jax.numpy as jnp
from jax import lax
from jax.experimental import pallas as pl
from jax.experimental.pallas import tpu as pltpu
from jax.experimental.pallas import tpu_sc as plsc

_B, _C, _N_IN = 4, 64, 196608
_N_OUT = _N_IN // 4
_TOT_IN = _B * _C * _N_IN
_TOT_OUT = _TOT_IN // 4
_NC, _NS = 2, 16
_NW = _NC * _NS
_IN_PER_W = _TOT_IN // _NW
_OUT_PER_W = _IN_PER_W // 4
_NSLOT = 4
_IC = 24576
_OC = _IC // 4
_NCHUNK = _IN_PER_W // _IC
_ROWS_PER_W = (_B * _C) // _NW
_CHUNKS_PER_ROW = _N_IN // _IC


@functools.partial(
    pl.kernel,
    out_type=jax.ShapeDtypeStruct((_B, _C, _N_OUT), jnp.float32),
    mesh=plsc.VectorSubcoreMesh(
        core_axis_name="c", subcore_axis_name="s",
        num_cores=_NC, num_subcores=_NS),
    scratch_types=(
        [pltpu.VMEM((_IC,), jnp.float32)] * _NSLOT
        + [pltpu.VMEM((_OC + 16,), jnp.float32)] * _NSLOT
        + [pltpu.SemaphoreType.DMA((_NSLOT,)),
           pltpu.SemaphoreType.DMA((_NSLOT,))]
    ),
    compiler_params=pltpu.CompilerParams(needs_layout_passes=False),
)
def _down(x_hbm, y_hbm, *bufs):
    ibuf = bufs[:_NSLOT]
    obuf = bufs[_NSLOT:2 * _NSLOT]
    isem, osem = bufs[2 * _NSLOT], bufs[2 * _NSLOT + 1]
    cid = lax.axis_index("c")
    sid = lax.axis_index("s")
    wid = sid * _NC + cid
    row0 = wid * _ROWS_PER_W
    lane = lax.iota(jnp.int32, 16)
    idx_r = tuple(lane * 4 + (lane // 4 + r) % 4 for r in range(4))

    def in_slice(g):
        row = row0 + g // _CHUNKS_PER_ROW
        p = g % _CHUNKS_PER_ROW
        return x_hbm.at[row // _C, row % _C, pl.ds(p * _IC, _IC)]

    def out_slice(g):
        row = row0 + g // _CHUNKS_PER_ROW
        p = g % _CHUNKS_PER_ROW
        return y_hbm.at[row // _C, row % _C, pl.ds(p * _OC, _OC)]

    for s in range(_NSLOT - 1):
        pltpu.async_copy(in_slice(s), ibuf[s], isem.at[s])

    @pl.loop(0, _NCHUNK // _NSLOT)
    def _outer(gg):
        for b in range(_NSLOT):
            g = gg * _NSLOT + b
            pf = (b + _NSLOT - 1) % _NSLOT

            @pl.when(g + _NSLOT - 1 < _NCHUNK)
            def _():
                pltpu.async_copy(
                    in_slice(g + _NSLOT - 1), ibuf[pf], isem.at[pf])

            pltpu.make_async_copy(in_slice(g), ibuf[b], isem.at[b]).wait()

            @pl.when(g >= _NSLOT)
            def _():
                pltpu.make_async_copy(
                    obuf[b].at[pl.ds(0, _OC)], out_slice(g - _NSLOT),
                    osem.at[b]).wait()

            ib = ibuf[b]
            ob = obuf[b]

            @plsc.parallel_loop(0, _OC // 16, unroll=8)
            def _step(v):
                base = v * 64
                v0 = plsc.load_gather(ib, [base + idx_r[0]])
                v1 = plsc.load_gather(ib, [base + idx_r[1]])
                v2 = plsc.load_gather(ib, [base + idx_r[2]])
                v3 = plsc.load_gather(ib, [base + idx_r[3]])
                m = jnp.maximum(jnp.maximum(v0, v1), jnp.maximum(v2, v3))
                ob[pl.ds(v * 16, 16)] = m

            pltpu.async_copy(
                obuf[b].at[pl.ds(0, _OC)], out_slice(g), osem.at[b])

    for b in range(_NSLOT):
        pltpu.make_async_copy(
            obuf[b].at[pl.ds(0, _OC)], out_slice(_NCHUNK - _NSLOT + b),
            osem.at[b]).wait()


def kernel(x):
    return _down(x)


# speedup vs baseline: 1.0058x; 1.0058x over previous
import functools

import jax
import jax.numpy as jnp
from jax import lax
from jax.experimental import pallas as pl
from jax.experimental.pallas import tpu as pltpu
from jax.experimental.pallas import tpu_sc as plsc

_B, _C, _N_IN = 4, 64, 196608
_N_OUT = _N_IN // 4
_TOT_IN = _B * _C * _N_IN
_TOT_OUT = _TOT_IN // 4
_NC, _NS = 2, 16
_NW = _NC * _NS
_IN_PER_W = _TOT_IN // _NW
_OUT_PER_W = _IN_PER_W // 4
_NSLOT = 3
_IC = 32768
_OC = _IC // 4
_NCHUNK = _IN_PER_W // _IC
_ROWS_PER_W = (_B * _C) // _NW
_CHUNKS_PER_ROW = _N_IN // _IC


@functools.partial(
    pl.kernel,
    out_type=jax.ShapeDtypeStruct((_B, _C, _N_OUT), jnp.float32),
    mesh=plsc.VectorSubcoreMesh(
        core_axis_name="c", subcore_axis_name="s",
        num_cores=_NC, num_subcores=_NS),
    scratch_types=(
        [pltpu.VMEM((_IC,), jnp.float32)] * _NSLOT
        + [pltpu.VMEM((_OC + 16,), jnp.float32)] * _NSLOT
        + [pltpu.SemaphoreType.DMA((_NSLOT,)),
           pltpu.SemaphoreType.DMA((_NSLOT,))]
    ),
    compiler_params=pltpu.CompilerParams(needs_layout_passes=False),
)
def _down(x_hbm, y_hbm, *bufs):
    ibuf = bufs[:_NSLOT]
    obuf = bufs[_NSLOT:2 * _NSLOT]
    isem, osem = bufs[2 * _NSLOT], bufs[2 * _NSLOT + 1]
    cid = lax.axis_index("c")
    sid = lax.axis_index("s")
    wid = sid * _NC + cid
    row0 = wid * _ROWS_PER_W
    lane = lax.iota(jnp.int32, 16)
    idx_r = tuple(lane * 4 + (lane // 4 + r) % 4 for r in range(4))

    def in_slice(g):
        row = row0 + g // _CHUNKS_PER_ROW
        p = g % _CHUNKS_PER_ROW
        return x_hbm.at[row // _C, row % _C, pl.ds(p * _IC, _IC)]

    def out_slice(g):
        row = row0 + g // _CHUNKS_PER_ROW
        p = g % _CHUNKS_PER_ROW
        return y_hbm.at[row // _C, row % _C, pl.ds(p * _OC, _OC)]

    for s in range(_NSLOT - 1):
        pltpu.async_copy(in_slice(s), ibuf[s], isem.at[s])

    @pl.loop(0, _NCHUNK // _NSLOT)
    def _outer(gg):
        for b in range(_NSLOT):
            g = gg * _NSLOT + b
            pf = (b + _NSLOT - 1) % _NSLOT

            @pl.when(g + _NSLOT - 1 < _NCHUNK)
            def _():
                pltpu.async_copy(
                    in_slice(g + _NSLOT - 1), ibuf[pf], isem.at[pf])

            pltpu.make_async_copy(in_slice(g), ibuf[b], isem.at[b]).wait()

            @pl.when(g >= _NSLOT)
            def _():
                pltpu.make_async_copy(
                    obuf[b].at[pl.ds(0, _OC)], out_slice(g - _NSLOT),
                    osem.at[b]).wait()

            ib = ibuf[b]
            ob = obuf[b]

            @plsc.parallel_loop(0, _OC // 16, unroll=8)
            def _step(v):
                base = v * 64
                v0 = plsc.load_gather(ib, [base + idx_r[0]])
                v1 = plsc.load_gather(ib, [base + idx_r[1]])
                v2 = plsc.load_gather(ib, [base + idx_r[2]])
                v3 = plsc.load_gather(ib, [base + idx_r[3]])
                m = jnp.maximum(jnp.maximum(v0, v1), jnp.maximum(v2, v3))
                ob[pl.ds(v * 16, 16)] = m

            pltpu.async_copy(
                obuf[b].at[pl.ds(0, _OC)], out_slice(g), osem.at[b])

    for b in range(_NSLOT):
        pltpu.make_async_copy(
            obuf[b].at[pl.ds(0, _OC)], out_slice(_NCHUNK - _NSLOT + b),
            osem.at[b]).wait()


def kernel(x):
    return _down(x)
